# baseline (speedup 1.0000x reference)
import functools

import jax
import jax.numpy as jnp
from jax import lax
from jax.experimental import pallas as pl
from jax.experimental.pallas import tpu as pltpu
from jax.experimental.pallas import tpu_sc as plsc

_LANES = 128


def _sc_gather(table, idx_flat):
    info = plsc.get_sparse_core_info()
    nw = info.num_cores * info.num_subcores
    nr = idx_flat.shape[0] // _LANES
    rows_per_w = nr // nw
    mesh = plsc.VectorSubcoreMesh(core_axis_name="c", subcore_axis_name="s")

    @functools.partial(
        pl.kernel,
        mesh=mesh,
        out_type=jax.ShapeDtypeStruct((nr * _LANES,), jnp.int32),
        scratch_types=[
            pltpu.VMEM((rows_per_w * _LANES,), jnp.int32),
            pltpu.VMEM((rows_per_w * _LANES,), jnp.int32),
            pltpu.SemaphoreType.DMA,
        ],
    )
    def gather_kernel(table_hbm, idx_hbm, out_hbm, idx_v, vals_v, sem):
        wid = lax.axis_index("s") * info.num_cores + lax.axis_index("c")
        n_per_w = rows_per_w * _LANES
        base = wid * n_per_w
        pltpu.sync_copy(idx_hbm.at[pl.ds(base, n_per_w)], idx_v)
        pltpu.async_copy(table_hbm.at[idx_v], vals_v, sem).wait()
        pltpu.sync_copy(vals_v, out_hbm.at[pl.ds(base, n_per_w)])

    return gather_kernel(table, idx_flat)


def _tc_loss(gathered2d, idx2d, t2d, w2d, n_batches):
    nr = gathered2d.shape[0]
    rows_per_b = nr // n_batches

    def body(g_ref, i_ref, t_ref, w_ref, out_ref):
        b = pl.program_id(0)
        words = g_ref[...]
        parity_hi = (i_ref[...] & 1) == 1
        bits = jnp.where(parity_hi, words & jnp.int32(-65536), words << 16)
        x = lax.bitcast_convert_type(bits, jnp.float32)
        t = t_ref[...]
        w = w_ref[...]
        loss = jnp.maximum(x, 0.0) - x * t + jnp.log1p(jnp.exp(-jnp.abs(x)))
        num = jnp.sum(loss * w)
        den = jnp.sum(t * w)

        @pl.when(b == 0)
        def _():
            out_ref[0, 0] = 0.0

        out_ref[0, 0] += num / (den * n_batches)

    out = pl.pallas_call(
        body,
        grid=(n_batches,),
        in_specs=[
            pl.BlockSpec((rows_per_b, _LANES), lambda b: (b, 0)),
            pl.BlockSpec((rows_per_b, _LANES), lambda b: (b, 0)),
            pl.BlockSpec((rows_per_b, _LANES), lambda b: (b, 0)),
            pl.BlockSpec((rows_per_b, _LANES), lambda b: (b, 0)),
        ],
        out_specs=pl.BlockSpec(memory_space=pltpu.SMEM),
        out_shape=jax.ShapeDtypeStruct((1, 1), jnp.float32),
    )(gathered2d, idx2d, t2d, w2d)
    return out[0, 0]


def kernel(voxel_occupancy, voxels_in_ray, occupany_of_voxels_in_ray, norm_dist):
    b, _, z, y, x = voxel_occupancy.shape
    v = z * y * x
    r = voxels_in_ray.shape[1]
    table_words = lax.bitcast_convert_type(
        voxel_occupancy.astype(jnp.bfloat16).reshape(b * v // 2, 2), jnp.int32
    )
    idx = voxels_in_ray.astype(jnp.int32) + (jnp.arange(b, dtype=jnp.int32) * v)[:, None]
    idx_flat = idx.reshape(-1)
    word_idx = idx_flat >> 1
    gathered2d = _sc_gather(table_words, word_idx).reshape(-1, _LANES)
    idx2d = idx_flat.reshape(-1, _LANES)
    t2d = occupany_of_voxels_in_ray.reshape(-1, _LANES)
    w2d = norm_dist.reshape(-1, _LANES)
    return _tc_loss(gathered2d, idx2d, t2d, w2d, b)


# speedup vs baseline: 1.3727x; 1.3727x over previous
import functools

import jax
import jax.numpy as jnp
from jax import lax
from jax.experimental import pallas as pl
from jax.experimental.pallas import tpu as pltpu
from jax.experimental.pallas import tpu_sc as plsc

_LANES = 128


def _sc_gather(table, idx_flat):
    info = plsc.get_sparse_core_info()
    nw = info.num_cores * info.num_subcores
    nr = idx_flat.shape[0] // _LANES
    rows_per_w = nr // nw
    mesh = plsc.VectorSubcoreMesh(core_axis_name="c", subcore_axis_name="s")

    @functools.partial(
        pl.kernel,
        mesh=mesh,
        out_type=jax.ShapeDtypeStruct((nr * _LANES,), jnp.int32),
        scratch_types=[
            pltpu.VMEM((rows_per_w * _LANES,), jnp.int32),
            pltpu.VMEM((rows_per_w * _LANES,), jnp.int32),
            pltpu.SemaphoreType.DMA,
        ],
    )
    def gather_kernel(table_hbm, idx_hbm, out_hbm, idx_v, vals_v, sem):
        wid = lax.axis_index("s") * info.num_cores + lax.axis_index("c")
        n_per_w = rows_per_w * _LANES
        base = wid * n_per_w
        pltpu.sync_copy(idx_hbm.at[pl.ds(base, n_per_w)], idx_v)
        pltpu.async_copy(table_hbm.at[idx_v], vals_v, sem).wait()
        pltpu.sync_copy(vals_v, out_hbm.at[pl.ds(base, n_per_w)])

    return gather_kernel(table, idx_flat)


def _tc_loss(gathered2d, idx2d, t2d, w2d, n_batches):
    nr = gathered2d.shape[0]
    rows_per_b = nr // n_batches

    def body(g_ref, i_ref, t_ref, w_ref, out_ref):
        b = pl.program_id(0)
        words = g_ref[...]
        parity_hi = (i_ref[...] & 1) == 1
        bits = jnp.where(parity_hi, words & jnp.int32(-65536), words << 16)
        x = lax.bitcast_convert_type(bits, jnp.float32)
        t = t_ref[...]
        w = w_ref[...]
        loss = jnp.maximum(x, 0.0) - x * t + jnp.log1p(jnp.exp(-jnp.abs(x)))
        num = jnp.sum(loss * w)
        den = jnp.sum(t * w)

        @pl.when(b == 0)
        def _():
            out_ref[0, 0] = 0.0

        out_ref[0, 0] += num / (den * n_batches)

    out = pl.pallas_call(
        body,
        grid=(n_batches,),
        in_specs=[
            pl.BlockSpec((rows_per_b, _LANES), lambda b: (b, 0)),
            pl.BlockSpec((rows_per_b, _LANES), lambda b: (b, 0)),
            pl.BlockSpec((rows_per_b, _LANES), lambda b: (b, 0)),
            pl.BlockSpec((rows_per_b, _LANES), lambda b: (b, 0)),
        ],
        out_specs=pl.BlockSpec(memory_space=pltpu.SMEM),
        out_shape=jax.ShapeDtypeStruct((1, 1), jnp.float32),
    )(gathered2d, idx2d, t2d, w2d)
    return out[0, 0]


def kernel(voxel_occupancy, voxels_in_ray, occupany_of_voxels_in_ray, norm_dist):
    b, _, z, y, x = voxel_occupancy.shape
    v = z * y * x
    r = voxels_in_ray.shape[1]
    flat_bits = lax.bitcast_convert_type(voxel_occupancy.reshape(b * v), jnp.int32)
    lo_bits, hi_bits = flat_bits[0::2], flat_bits[1::2]

    def _rne16(bits):
        lsb = lax.shift_right_logical(bits, 16) & 1
        return lax.shift_right_logical(bits + 0x7FFF + lsb, 16)

    table_words = lax.shift_left(_rne16(hi_bits), 16) | _rne16(lo_bits)
    idx = voxels_in_ray.astype(jnp.int32) + (jnp.arange(b, dtype=jnp.int32) * v)[:, None]
    idx_flat = idx.reshape(-1)
    word_idx = idx_flat >> 1
    gathered2d = _sc_gather(table_words, word_idx).reshape(-1, _LANES)
    idx2d = idx_flat.reshape(-1, _LANES)
    t2d = occupany_of_voxels_in_ray.reshape(-1, _LANES)
    w2d = norm_dist.reshape(-1, _LANES)
    return _tc_loss(gathered2d, idx2d, t2d, w2d, b)


# speedup vs baseline: 24.0259x; 17.5023x over previous
import functools

import jax
import jax.numpy as jnp
from jax import lax
from jax.experimental import pallas as pl
from jax.experimental.pallas import tpu as pltpu
from jax.experimental.pallas import tpu_sc as plsc

_LANES = 128


def _sc_gather(table, idx_flat):
    info = plsc.get_sparse_core_info()
    nw = info.num_cores * info.num_subcores
    nr = idx_flat.shape[0] // _LANES
    rows_per_w = nr // nw
    mesh = plsc.VectorSubcoreMesh(core_axis_name="c", subcore_axis_name="s")

    @functools.partial(
        pl.kernel,
        mesh=mesh,
        out_type=jax.ShapeDtypeStruct((nr * _LANES,), jnp.float32),
        scratch_types=[
            pltpu.VMEM((rows_per_w * _LANES,), jnp.int32),
            pltpu.VMEM((rows_per_w * _LANES,), jnp.float32),
            pltpu.SemaphoreType.DMA,
        ],
    )
    def gather_kernel(table_hbm, idx_hbm, out_hbm, idx_v, vals_v, sem):
        wid = lax.axis_index("s") * info.num_cores + lax.axis_index("c")
        n_per_w = rows_per_w * _LANES
        base = wid * n_per_w
        pltpu.sync_copy(idx_hbm.at[pl.ds(base, n_per_w)], idx_v)
        pltpu.async_copy(table_hbm.at[idx_v], vals_v, sem).wait()
        pltpu.sync_copy(vals_v, out_hbm.at[pl.ds(base, n_per_w)])

    return gather_kernel(table, idx_flat)


def _tc_loss(g01, g23, t2d, w2d, n_batches):
    nr = t2d.shape[0]
    rows_per_b = nr // n_batches
    half = n_batches // 2

    def body(ga_ref, gb_ref, t_ref, w_ref, out_ref):
        b = pl.program_id(0)
        x = jnp.where(b < half, ga_ref[...], gb_ref[...])
        t = t_ref[...]
        w = w_ref[...]
        loss = jnp.maximum(x, 0.0) - x * t + jnp.log1p(jnp.exp(-jnp.abs(x)))
        num = jnp.sum(loss * w)
        den = jnp.sum(t * w)

        @pl.when(b == 0)
        def _():
            out_ref[0, 0] = 0.0

        out_ref[0, 0] += num / (den * n_batches)

    out = pl.pallas_call(
        body,
        grid=(n_batches,),
        in_specs=[
            pl.BlockSpec((rows_per_b, _LANES), lambda b: (jnp.minimum(b, half - 1), 0)),
            pl.BlockSpec((rows_per_b, _LANES), lambda b: (jnp.maximum(b - half, 0), 0)),
            pl.BlockSpec((rows_per_b, _LANES), lambda b: (b, 0)),
            pl.BlockSpec((rows_per_b, _LANES), lambda b: (b, 0)),
        ],
        out_specs=pl.BlockSpec(memory_space=pltpu.SMEM),
        out_shape=jax.ShapeDtypeStruct((1, 1), jnp.float32),
    )(g01, g23, t2d, w2d)
    return out[0, 0]


def kernel(voxel_occupancy, voxels_in_ray, occupany_of_voxels_in_ray, norm_dist):
    b, _, z, y, x = voxel_occupancy.shape
    v = z * y * x
    half = b // 2
    idx32 = voxels_in_ray.astype(jnp.int32)
    off = (jnp.arange(half, dtype=jnp.int32) * v)[:, None]
    table01 = voxel_occupancy[:half].reshape(half * v)
    table23 = voxel_occupancy[half:].reshape(half * v)
    idx01 = (idx32[:half] + off).reshape(-1)
    idx23 = (idx32[half:] + off).reshape(-1)
    g01 = _sc_gather(table01, idx01).reshape(-1, _LANES)
    g23 = _sc_gather(table23, idx23).reshape(-1, _LANES)
    t2d = occupany_of_voxels_in_ray.reshape(-1, _LANES)
    w2d = norm_dist.reshape(-1, _LANES)
    return _tc_loss(g01, g23, t2d, w2d, b)


# speedup vs baseline: 30.6712x; 1.2766x over previous
import functools

import jax
import jax.numpy as jnp
from jax import lax
from jax.experimental import pallas as pl
from jax.experimental.pallas import tpu as pltpu
from jax.experimental.pallas import tpu_sc as plsc

_LANES = 128


def _sc_gather(table, idx_flat):
    info = plsc.get_sparse_core_info()
    nw = info.num_cores * info.num_subcores
    nr = idx_flat.shape[0] // _LANES
    rows_per_w = nr // nw
    mesh = plsc.VectorSubcoreMesh(core_axis_name="c", subcore_axis_name="s")

    @functools.partial(
        pl.kernel,
        mesh=mesh,
        out_type=jax.ShapeDtypeStruct((nr * _LANES,), jnp.float32),
        scratch_types=[
            pltpu.VMEM((rows_per_w * _LANES,), jnp.int32),
            pltpu.VMEM((rows_per_w * _LANES,), jnp.float32),
            pltpu.SemaphoreType.DMA,
        ],
    )
    def gather_kernel(table_hbm, idx_hbm, out_hbm, idx_v, vals_v, sem):
        wid = lax.axis_index("s") * info.num_cores + lax.axis_index("c")
        n_per_w = rows_per_w * _LANES
        base = wid * n_per_w
        pltpu.sync_copy(idx_hbm.at[pl.ds(base, n_per_w)], idx_v)
        pltpu.async_copy(table_hbm.at[idx_v], vals_v, sem).wait()
        pltpu.sync_copy(vals_v, out_hbm.at[pl.ds(base, n_per_w)])

    return gather_kernel(table, idx_flat)


def _tc_loss(gathered2d, t2d, w2d, n_batches):
    nr = gathered2d.shape[0]
    rows_per_b = nr // n_batches

    def body(g_ref, t_ref, w_ref, out_ref):
        b = pl.program_id(0)
        x = g_ref[...]
        t = t_ref[...]
        w = w_ref[...]
        loss = jnp.maximum(x, 0.0) - x * t + jnp.log1p(jnp.exp(-jnp.abs(x)))
        num = jnp.sum(loss * w)
        den = jnp.sum(t * w)

        @pl.when(b == 0)
        def _():
            out_ref[0, 0] = 0.0

        out_ref[0, 0] += num / (den * n_batches)

    out = pl.pallas_call(
        body,
        grid=(n_batches,),
        in_specs=[
            pl.BlockSpec((rows_per_b, _LANES), lambda b: (b, 0)),
            pl.BlockSpec((rows_per_b, _LANES), lambda b: (b, 0)),
            pl.BlockSpec((rows_per_b, _LANES), lambda b: (b, 0)),
        ],
        out_specs=pl.BlockSpec(memory_space=pltpu.SMEM),
        out_shape=jax.ShapeDtypeStruct((1, 1), jnp.float32),
    )(gathered2d, t2d, w2d)
    return out[0, 0]


def kernel(voxel_occupancy, voxels_in_ray, occupany_of_voxels_in_ray, norm_dist):
    b, _, z, y, x = voxel_occupancy.shape
    v = z * y * x
    r = voxels_in_ray.shape[1]
    table = voxel_occupancy.reshape(b * v)
    idx = voxels_in_ray.astype(jnp.int32) + (jnp.arange(b, dtype=jnp.int32) * v)[:, None]
    idx_flat = idx.reshape(-1)
    gathered2d = _sc_gather(table, idx_flat).reshape(-1, _LANES)
    t2d = occupany_of_voxels_in_ray.reshape(-1, _LANES)
    w2d = norm_dist.reshape(-1, _LANES)
    return _tc_loss(gathered2d, t2d, w2d, b)
